# baseline (speedup 1.0000x reference)
import functools

import jax
import jax.numpy as jnp
from jax import lax
from jax.experimental import pallas as pl
from jax.experimental.pallas import tpu as pltpu
from jax.experimental.pallas import tpu_sc as plsc

B = 16384
K = 5
D = 64
CB = 128
NC, NS = 2, 16
NW = NC * NS
BPW = B // NW
NCH = BPW // CB
NG = CB // 16

_mesh = plsc.VectorSubcoreMesh(core_axis_name="c", subcore_axis_name="s")


@functools.partial(
    pl.kernel,
    out_type=jax.ShapeDtypeStruct((B * K,), jnp.float32),
    mesh=_mesh,
    scratch_types=[
        pltpu.VMEM((2, CB), jnp.int32),
        pltpu.VMEM((2, CB * K), jnp.int32),
        pltpu.VMEM((2, K, CB), jnp.int32),
        pltpu.VMEM((2, CB, D), jnp.float32),
        pltpu.VMEM((2, K, CB, D), jnp.float32),
        pltpu.VMEM((BPW * K,), jnp.float32),
        pltpu.SemaphoreType.DMA,
        pltpu.SemaphoreType.DMA,
    ],
    compiler_params=pltpu.CompilerParams(needs_layout_passes=False,
                                         use_tc_tiling_on_sc=False),
)
def _sc_skipgram(tgt_hbm, ctx_hbm, ttab_hbm, otab_hbm, out_hbm,
                 tidx_v, cidx_v, kidx_v, trows_v, crows_v, outb_v,
                 sem0, sem1):
    wid = lax.axis_index("s") * NC + lax.axis_index("c")
    b0w = wid * BPW
    sems = [sem0, sem1]
    descs = [None, None]

    def fire(c):
        s = c % 2
        b0 = b0w + c * CB
        pltpu.sync_copy(tgt_hbm.at[pl.ds(b0, CB)], tidx_v.at[s])
        pltpu.sync_copy(ctx_hbm.at[pl.ds(b0 * K, CB * K)], cidx_v.at[s])
        for g in range(NG):
            rowg5 = (lax.iota(jnp.int32, 16) + g * 16) * K
            for k in range(K):
                kidx_v[s, k, pl.ds(g * 16, 16)] = plsc.load_gather(
                    cidx_v.at[s], [rowg5 + k])
        ds = [pltpu.async_copy(ttab_hbm.at[tidx_v.at[s]], trows_v.at[s],
                               sems[s])]
        for k in range(K):
            ds.append(pltpu.async_copy(otab_hbm.at[kidx_v.at[s, k]],
                                       crows_v.at[s, k], sems[s]))
        descs[s] = ds

    def compute(c):
        s = c % 2
        trows = trows_v.at[s]
        crows = crows_v.at[s]
        for g in range(NG):
            rowg = lax.iota(jnp.int32, 16) + g * 16

            def body(d, accs):
                dvec = lax.broadcast(d, (16,))
                tcol = plsc.load_gather(trows, [rowg, dvec])
                return tuple(
                    accs[k] + tcol * plsc.load_gather(crows.at[k], [rowg, dvec])
                    for k in range(K))

            accs = lax.fori_loop(
                0, D, body,
                tuple(jnp.zeros((16,), jnp.float32) for _ in range(K)))
            pbase = (rowg + c * CB) * K
            for k in range(K):
                plsc.store_scatter(outb_v, [pbase + k], accs[k])

    fire(0)
    for c in range(NCH):
        if c + 1 < NCH:
            fire(c + 1)
        for d in descs[c % 2]:
            d.wait()
        compute(c)
    pltpu.sync_copy(outb_v, out_hbm.at[pl.ds(b0w * K, BPW * K)])


def kernel(target, context, target_table, output_table):
    ctx_flat = context.astype(jnp.int32).reshape(B * K)
    out_flat = _sc_skipgram(target.astype(jnp.int32), ctx_flat,
                            target_table, output_table)
    return out_flat.reshape(B, K)
